# Initial kernel scaffold; baseline (speedup 1.0000x reference)
import jax
import jax.numpy as jnp
from jax.experimental import pallas as pl


def kernel(x, params, edge_index, batch):
    raise NotImplementedError("write your pallas kernel here")


# speedup vs baseline: 7.4664x; 7.4664x over previous
import jax
import jax.numpy as jnp
from jax import lax
from jax.experimental import pallas as pl
from jax.experimental.pallas import tpu as pltpu
from jax.experimental.pallas import tpu_sc as plsc

N = 10000
E = 320000
D = 128
H = 64
G = 128

NC = 2
NS = 16
NW = NC * NS
CH = 80
NCHUNK = E // (NW * CH)
RPT = 640
NPAD = NS * RPT

_F32 = jnp.float32
_HIGH = lax.Precision.HIGHEST


def _sc_agg_body(q_hbm, src_hbm, dst_hbm, zeros_hbm, out_hbm,
                 src_v, dst_v, rows_v, acc_sh, sem):
    c = lax.axis_index("c")
    s = lax.axis_index("s")
    wid = c * NS + s

    pltpu.sync_copy(zeros_hbm, acc_sh.at[pl.ds(s * RPT, RPT)])
    pltpu.sync_copy(src_hbm.at[wid], src_v)
    pltpu.sync_copy(dst_hbm.at[wid], dst_v)
    plsc.subcore_barrier()

    def chunk(j, carry):
        pltpu.async_copy(q_hbm.at[src_v.at[j]], rows_v, sem).wait()
        pltpu.sync_copy(rows_v, acc_sh.at[dst_v.at[j]], add=True)
        return carry

    lax.fori_loop(0, NCHUNK, chunk, 0)

    plsc.subcore_barrier()
    pltpu.sync_copy(acc_sh.at[pl.ds(s * RPT, RPT)],
                    out_hbm.at[c, pl.ds(s * RPT, RPT)])


_SC_AGG_CACHE = []


def _sc_agg(*args):
    if not _SC_AGG_CACHE:
        _SC_AGG_CACHE.append(pl.kernel(
            _sc_agg_body,
            out_type=jax.ShapeDtypeStruct((NC, NPAD, H), _F32),
            mesh=plsc.VectorSubcoreMesh(core_axis_name="c",
                                        subcore_axis_name="s",
                                        num_cores=NC, num_subcores=NS),
            scratch_types=[
                pltpu.VMEM((NCHUNK, CH), jnp.int32),
                pltpu.VMEM((NCHUNK, CH), jnp.int32),
                pltpu.VMEM((CH, H), _F32),
                pltpu.VMEM_SHARED((NPAD, H), _F32),
                pltpu.SemaphoreType.DMA,
            ],
            compiler_params=pltpu.CompilerParams(use_tc_tiling_on_sc=False),
        ))
    return _SC_AGG_CACHE[0](*args)


def _mm_body(x_ref, w_ref, o_ref):
    o_ref[...] = lax.dot(x_ref[...], w_ref[...], precision=_HIGH,
                         preferred_element_type=_F32)


_mm = pl.pallas_call(
    _mm_body, out_shape=jax.ShapeDtypeStruct((N, H), _F32))


def _bn_relu(z, g, be):
    m = jnp.sum(z, axis=0, keepdims=True) * (1.0 / N)
    d = z - m
    v = jnp.sum(d * d, axis=0, keepdims=True) * (1.0 / N)
    return jnp.maximum(d * lax.rsqrt(v + 1e-5) * g + be, 0.0)


def _mid_body(q_ref, agg_ref, b_ref, g_ref, be_ref, w_ref, o_ref):
    z = q_ref[...] + agg_ref[0, :N, :] + agg_ref[1, :N, :] + b_ref[...]
    h = _bn_relu(z, g_ref[...], be_ref[...])
    o_ref[...] = lax.dot(h, w_ref[...], precision=_HIGH,
                         preferred_element_type=_F32)


_mid = pl.pallas_call(
    _mid_body, out_shape=jax.ShapeDtypeStruct((N, H), _F32))


def _fin_body(q_ref, agg_ref, b_ref, g_ref, be_ref, batch_ref,
              w1_ref, b1_ref, w2_ref, b2_ref, o_ref):
    z = q_ref[...] + agg_ref[0, :N, :] + agg_ref[1, :N, :] + b_ref[...]
    h = _bn_relu(z, g_ref[...], be_ref[...])
    ids = batch_ref[...]
    iota = lax.broadcasted_iota(jnp.int32, (N, G), 1)
    onehot = (ids == iota).astype(_F32)
    dn = (((0,), (0,)), ((), ()))
    sums = lax.dot_general(onehot, h, dn, precision=_HIGH,
                           preferred_element_type=_F32)
    ones = jnp.ones((N, 1), _F32)
    cnt = lax.dot_general(onehot, ones, dn, precision=_HIGH,
                          preferred_element_type=_F32)
    pooled = sums / jnp.maximum(cnt, 1.0)
    h2 = jnp.maximum(
        lax.dot(pooled, w1_ref[...], precision=_HIGH,
                preferred_element_type=_F32) + b1_ref[...], 0.0)
    logit = jnp.sum(h2 * w2_ref[...], axis=1, keepdims=True) + b2_ref[...]
    o_ref[...] = jax.nn.sigmoid(logit)


_fin = pl.pallas_call(
    _fin_body, out_shape=jax.ShapeDtypeStruct((G, 1), _F32))


def kernel(x, params, edge_index, batch):
    src = edge_index[:, 0].reshape(NW, NCHUNK, CH)
    dst = edge_index[:, 1].reshape(NW, NCHUNK, CH)
    zeros = jnp.zeros((RPT, H), _F32)
    batch2d = batch.reshape(N, 1)

    q = _mm(x, params['W1'])
    for i in range(1, 6):
        aggs = _sc_agg(q, src, dst, zeros)
        b = params['b%d' % i].reshape(1, H)
        g = params['g%d' % i].reshape(1, H)
        be = params['be%d' % i].reshape(1, H)
        if i < 5:
            q = _mid(q, aggs, b, g, be, params['W%d' % (i + 1)])
        else:
            out = _fin(q, aggs, b, g, be, batch2d,
                       params['fc1_W'], params['fc1_b'].reshape(1, H),
                       params['fc2_W'].reshape(1, H),
                       params['fc2_b'].reshape(1, 1))
    return out.reshape(G)


# speedup vs baseline: 13.8123x; 1.8499x over previous
import jax
import jax.numpy as jnp
from jax import lax
from jax.experimental import pallas as pl
from jax.experimental.pallas import tpu as pltpu
from jax.experimental.pallas import tpu_sc as plsc

N = 10000
E = 320000
D = 128
H = 64
G = 128

NC = 2
NS = 16
NW = NC * NS
CH = 80
NCHUNK = E // (NW * CH)
RPT = 640
NPAD = NS * RPT

_F32 = jnp.float32
_HIGH = lax.Precision.HIGHEST


NBUF = 5
NOUTER = NCHUNK // NBUF


def _sc_agg_body(q_hbm, src_hbm, dst_hbm, zeros_hbm, out_hbm,
                 src_v, dst_v, rows_v, acc_sh, sem_g, sem_s):
    c = lax.axis_index("c")
    s = lax.axis_index("s")
    wid = c * NS + s

    pltpu.sync_copy(zeros_hbm, acc_sh.at[pl.ds(s * RPT, RPT)])
    pltpu.sync_copy(src_hbm.at[wid], src_v)
    pltpu.sync_copy(dst_hbm.at[wid], dst_v)
    plsc.subcore_barrier()

    def _drain_scatter(b):
        pltpu.make_async_copy(q_hbm.at[pl.ds(0, CH)], rows_v.at[b],
                              sem_s).wait()

    def outer(i, carry):
        gathers = []
        for b in range(NBUF):
            @pl.when(i > 0)
            def _(b=b):
                _drain_scatter(b)
            gathers.append(pltpu.async_copy(
                q_hbm.at[src_v.at[i * NBUF + b]], rows_v.at[b], sem_g))
        for b in range(NBUF):
            gathers[b].wait()
            pltpu.async_copy(rows_v.at[b], acc_sh.at[dst_v.at[i * NBUF + b]],
                             sem_s, add=True)
        return carry

    lax.fori_loop(0, NOUTER, outer, 0)
    for b in range(NBUF):
        _drain_scatter(b)

    plsc.subcore_barrier()
    pltpu.sync_copy(acc_sh.at[pl.ds(s * RPT, RPT)],
                    out_hbm.at[c, pl.ds(s * RPT, RPT)])


_SC_AGG_CACHE = []


def _sc_agg(*args):
    if not _SC_AGG_CACHE:
        _SC_AGG_CACHE.append(pl.kernel(
            _sc_agg_body,
            out_type=jax.ShapeDtypeStruct((NC, NPAD, H), _F32),
            mesh=plsc.VectorSubcoreMesh(core_axis_name="c",
                                        subcore_axis_name="s",
                                        num_cores=NC, num_subcores=NS),
            scratch_types=[
                pltpu.VMEM((NCHUNK, CH), jnp.int32),
                pltpu.VMEM((NCHUNK, CH), jnp.int32),
                pltpu.VMEM((NBUF, CH, H), _F32),
                pltpu.VMEM_SHARED((NPAD, H), _F32),
                pltpu.SemaphoreType.DMA,
                pltpu.SemaphoreType.DMA,
            ],
            compiler_params=pltpu.CompilerParams(use_tc_tiling_on_sc=False),
        ))
    return _SC_AGG_CACHE[0](*args)


def _mm_body(x_ref, w_ref, o_ref):
    o_ref[...] = lax.dot(x_ref[...], w_ref[...], precision=_HIGH,
                         preferred_element_type=_F32)


_mm = pl.pallas_call(
    _mm_body, out_shape=jax.ShapeDtypeStruct((N, H), _F32))


def _bn_relu(z, g, be):
    m = jnp.sum(z, axis=0, keepdims=True) * (1.0 / N)
    d = z - m
    v = jnp.sum(d * d, axis=0, keepdims=True) * (1.0 / N)
    return jnp.maximum(d * lax.rsqrt(v + 1e-5) * g + be, 0.0)


def _mid_body(q_ref, agg_ref, b_ref, g_ref, be_ref, w_ref, o_ref):
    z = q_ref[...] + agg_ref[0, :N, :] + agg_ref[1, :N, :] + b_ref[...]
    h = _bn_relu(z, g_ref[...], be_ref[...])
    o_ref[...] = lax.dot(h, w_ref[...], precision=_HIGH,
                         preferred_element_type=_F32)


_mid = pl.pallas_call(
    _mid_body, out_shape=jax.ShapeDtypeStruct((N, H), _F32))


def _fin_body(q_ref, agg_ref, b_ref, g_ref, be_ref, batch_ref,
              w1_ref, b1_ref, w2_ref, b2_ref, o_ref):
    z = q_ref[...] + agg_ref[0, :N, :] + agg_ref[1, :N, :] + b_ref[...]
    h = _bn_relu(z, g_ref[...], be_ref[...])
    ids = batch_ref[...]
    iota = lax.broadcasted_iota(jnp.int32, (N, G), 1)
    onehot = (ids == iota).astype(_F32)
    dn = (((0,), (0,)), ((), ()))
    sums = lax.dot_general(onehot, h, dn, precision=_HIGH,
                           preferred_element_type=_F32)
    ones = jnp.ones((N, 1), _F32)
    cnt = lax.dot_general(onehot, ones, dn, precision=_HIGH,
                          preferred_element_type=_F32)
    pooled = sums / jnp.maximum(cnt, 1.0)
    h2 = jnp.maximum(
        lax.dot(pooled, w1_ref[...], precision=_HIGH,
                preferred_element_type=_F32) + b1_ref[...], 0.0)
    logit = jnp.sum(h2 * w2_ref[...], axis=1, keepdims=True) + b2_ref[...]
    o_ref[...] = jax.nn.sigmoid(logit)


_fin = pl.pallas_call(
    _fin_body, out_shape=jax.ShapeDtypeStruct((G, 1), _F32))


def kernel(x, params, edge_index, batch):
    src = edge_index[:, 0].reshape(NW, NCHUNK, CH)
    dst = edge_index[:, 1].reshape(NW, NCHUNK, CH)
    zeros = jnp.zeros((RPT, H), _F32)
    batch2d = batch.reshape(N, 1)

    q = _mm(x, params['W1'])
    for i in range(1, 6):
        aggs = _sc_agg(q, src, dst, zeros)
        b = params['b%d' % i].reshape(1, H)
        g = params['g%d' % i].reshape(1, H)
        be = params['be%d' % i].reshape(1, H)
        if i < 5:
            q = _mid(q, aggs, b, g, be, params['W%d' % (i + 1)])
        else:
            out = _fin(q, aggs, b, g, be, batch2d,
                       params['fc1_W'], params['fc1_b'].reshape(1, H),
                       params['fc2_W'].reshape(1, H),
                       params['fc2_b'].reshape(1, 1))
    return out.reshape(G)


# speedup vs baseline: 16.8190x; 1.2177x over previous
import jax
import jax.numpy as jnp
from jax import lax
from jax.experimental import pallas as pl
from jax.experimental.pallas import tpu as pltpu
from jax.experimental.pallas import tpu_sc as plsc

N = 10000
E = 320000
D = 128
H = 64
G = 128

NC = 2
NS = 16
NW = NC * NS
CH = 80
NCHUNK = E // (NW * CH)
RPT = 640
NPAD = NS * RPT
NH = N // 2
MROW = NPAD // 2

_F32 = jnp.float32
_PREC = lax.Precision.HIGHEST


NBUF = 5
NOUTER = NCHUNK // NBUF


def _sc_agg_body(q_hbm, src_hbm, dst_hbm, zeros_hbm, out_hbm,
                 src_v, dst_v, rows_v, acc_sh, sem_g, sem_s):
    c = lax.axis_index("c")
    s = lax.axis_index("s")
    wid = c * NS + s

    pltpu.sync_copy(zeros_hbm, acc_sh.at[pl.ds(s * RPT, RPT)])
    pltpu.sync_copy(src_hbm.at[wid], src_v)
    pltpu.sync_copy(dst_hbm.at[wid], dst_v)
    plsc.subcore_barrier()

    def _drain_scatter(b):
        pltpu.make_async_copy(q_hbm.at[pl.ds(0, CH)], rows_v.at[b],
                              sem_s).wait()

    def outer(i, carry):
        gathers = []
        for b in range(NBUF):
            @pl.when(i > 0)
            def _(b=b):
                _drain_scatter(b)
            gathers.append(pltpu.async_copy(
                q_hbm.at[src_v.at[i * NBUF + b]], rows_v.at[b], sem_g))
        for b in range(NBUF):
            gathers[b].wait()
            pltpu.async_copy(rows_v.at[b], acc_sh.at[dst_v.at[i * NBUF + b]],
                             sem_s, add=True)
        return carry

    lax.fori_loop(0, NOUTER, outer, 0)
    for b in range(NBUF):
        _drain_scatter(b)

    plsc.subcore_barrier()
    pltpu.sync_copy(acc_sh.at[pl.ds(s * RPT, RPT)],
                    out_hbm.at[c, pl.ds(s * RPT, RPT)])


_SC_AGG_CACHE = []


def _sc_agg(*args):
    if not _SC_AGG_CACHE:
        _SC_AGG_CACHE.append(pl.kernel(
            _sc_agg_body,
            out_type=jax.ShapeDtypeStruct((NC, NPAD, H), _F32),
            mesh=plsc.VectorSubcoreMesh(core_axis_name="c",
                                        subcore_axis_name="s",
                                        num_cores=NC, num_subcores=NS),
            scratch_types=[
                pltpu.VMEM((NCHUNK, CH), jnp.int32),
                pltpu.VMEM((NCHUNK, CH), jnp.int32),
                pltpu.VMEM((NBUF, CH, H), _F32),
                pltpu.VMEM_SHARED((NPAD, H), _F32),
                pltpu.SemaphoreType.DMA,
                pltpu.SemaphoreType.DMA,
            ],
            compiler_params=pltpu.CompilerParams(use_tc_tiling_on_sc=False),
        ))
    return _SC_AGG_CACHE[0](*args)


def _mm_body(x_ref, wl_ref, wr_ref, o_ref):
    xe = x_ref[0:N:2, :]
    xo = x_ref[1:N:2, :]
    o_ref[...] = (
        lax.dot(xe, wl_ref[...], precision=_PREC, preferred_element_type=_F32)
        + lax.dot(xo, wr_ref[...], precision=_PREC,
                  preferred_element_type=_F32))


_mm = pl.pallas_call(
    _mm_body, out_shape=jax.ShapeDtypeStruct((NH, D), _F32))


def _fold_mean(s):
    m = s[:, :H] + s[:, H:]
    return jnp.concatenate([m, m], axis=1) * (1.0 / N)


def _bn_relu_packed(z, g2, be2):
    m = _fold_mean(jnp.sum(z, axis=0, keepdims=True))
    d = z - m
    v = _fold_mean(jnp.sum(d * d, axis=0, keepdims=True))
    return jnp.maximum(d * lax.rsqrt(v + 1e-5) * g2 + be2, 0.0)


def _zsum(q_ref, agg_ref, b2_ref):
    return (q_ref[...] + agg_ref[0:NH, :] + agg_ref[MROW:MROW + NH, :]
            + b2_ref[...])


def _mid_body(q_ref, agg_ref, b2_ref, g2_ref, be2_ref, wbd_ref, o_ref):
    z = _zsum(q_ref, agg_ref, b2_ref)
    h = _bn_relu_packed(z, g2_ref[...], be2_ref[...])
    o_ref[...] = lax.dot(h, wbd_ref[...], precision=_PREC,
                         preferred_element_type=_F32)


_mid = pl.pallas_call(
    _mid_body, out_shape=jax.ShapeDtypeStruct((NH, D), _F32))


def _fin_body(q_ref, agg_ref, b2_ref, g2_ref, be2_ref, be_ref, bo_ref,
              w1_ref, b1_ref, w2_ref, b2b_ref, o_ref):
    z = _zsum(q_ref, agg_ref, b2_ref)
    h = _bn_relu_packed(z, g2_ref[...], be2_ref[...])
    iota = lax.broadcasted_iota(jnp.int32, (NH, G), 1)
    ohe = (be_ref[...] == iota).astype(_F32)
    oho = (bo_ref[...] == iota).astype(_F32)
    dn = (((0,), (0,)), ((), ()))
    se = lax.dot_general(ohe, h, dn, precision=_PREC,
                         preferred_element_type=_F32)
    so = lax.dot_general(oho, h, dn, precision=_PREC,
                         preferred_element_type=_F32)
    sums = se[:, :H] + so[:, H:]
    ones = jnp.ones((NH, 1), _F32)
    cnt = (lax.dot_general(ohe, ones, dn, precision=_PREC,
                           preferred_element_type=_F32)
           + lax.dot_general(oho, ones, dn, precision=_PREC,
                             preferred_element_type=_F32))
    pooled = sums / jnp.maximum(cnt, 1.0)
    h2 = jnp.maximum(
        lax.dot(pooled, w1_ref[...], precision=_PREC,
                preferred_element_type=_F32) + b1_ref[...], 0.0)
    logit = jnp.sum(h2 * w2_ref[...], axis=1, keepdims=True) + b2b_ref[...]
    o_ref[...] = jax.nn.sigmoid(logit)


_fin = pl.pallas_call(
    _fin_body, out_shape=jax.ShapeDtypeStruct((G, 1), _F32))


def _pack2(v):
    return jnp.concatenate([v, v]).reshape(1, D)


def _blockdiag(w):
    zb = jnp.zeros((H, H), _F32)
    return jnp.concatenate(
        [jnp.concatenate([w, zb], axis=1),
         jnp.concatenate([zb, w], axis=1)], axis=0)


def kernel(x, params, edge_index, batch):
    src = edge_index[:, 0].reshape(NW, NCHUNK, CH)
    dst = edge_index[:, 1].reshape(NW, NCHUNK, CH)
    zeros = jnp.zeros((RPT, H), _F32)
    bh = batch.reshape(NH, 2)
    be2d = bh[:, 0].reshape(NH, 1)
    bo2d = bh[:, 1].reshape(NH, 1)
    zpad = jnp.zeros((D, H), _F32)
    w1l = jnp.concatenate([params['W1'], zpad], axis=1)
    w1r = jnp.concatenate([zpad, params['W1']], axis=1)

    q = _mm(x, w1l, w1r)
    for i in range(1, 6):
        aggs = _sc_agg(q.reshape(N, H), src, dst, zeros)
        aggp = aggs.reshape(NC * MROW, D)
        b2 = _pack2(params['b%d' % i])
        g2 = _pack2(params['g%d' % i])
        be2 = _pack2(params['be%d' % i])
        if i < 5:
            q = _mid(q, aggp, b2, g2, be2,
                     _blockdiag(params['W%d' % (i + 1)]))
        else:
            out = _fin(q, aggp, b2, g2, be2, be2d, bo2d,
                       params['fc1_W'], params['fc1_b'].reshape(1, H),
                       params['fc2_W'].reshape(1, H),
                       params['fc2_b'].reshape(1, 1))
    return out.reshape(G)


# speedup vs baseline: 17.2538x; 1.0259x over previous
import jax
import jax.numpy as jnp
from jax import lax
from jax.experimental import pallas as pl
from jax.experimental.pallas import tpu as pltpu
from jax.experimental.pallas import tpu_sc as plsc

N = 10000
E = 320000
D = 128
H = 64
G = 128

NC = 2
NS = 16
NW = NC * NS
CH = 125
NCHUNK = E // (NW * CH)
RPT = 640
NPAD = NS * RPT
NH = N // 2
MROW = NPAD // 2

_F32 = jnp.float32
_PREC = lax.Precision.HIGHEST


NBUF = 5
NOUTER = NCHUNK // NBUF


def _sc_agg_body(q_hbm, src_hbm, dst_hbm, zeros_hbm, out_hbm,
                 src_v, dst_v, rows_v, acc_sh, sem_g, sem_s):
    c = lax.axis_index("c")
    s = lax.axis_index("s")
    wid = c * NS + s

    z = pltpu.async_copy(zeros_hbm, acc_sh.at[pl.ds(s * RPT, RPT)], sem_g)
    a = pltpu.async_copy(src_hbm.at[wid], src_v, sem_g)
    b = pltpu.async_copy(dst_hbm.at[wid], dst_v, sem_g)
    z.wait()
    a.wait()
    b.wait()
    plsc.subcore_barrier()

    def _drain_scatter(b):
        pltpu.make_async_copy(q_hbm.at[pl.ds(0, CH)], rows_v.at[b],
                              sem_s).wait()

    def outer(i, carry):
        gathers = []
        for b in range(NBUF):
            @pl.when(i > 0)
            def _(b=b):
                _drain_scatter(b)
            gathers.append(pltpu.async_copy(
                q_hbm.at[src_v.at[i * NBUF + b]], rows_v.at[b], sem_g))
        for b in range(NBUF):
            gathers[b].wait()
            pltpu.async_copy(rows_v.at[b], acc_sh.at[dst_v.at[i * NBUF + b]],
                             sem_s, add=True)
        return carry

    lax.fori_loop(0, NOUTER, outer, 0)
    for b in range(NBUF):
        _drain_scatter(b)

    plsc.subcore_barrier()
    pltpu.sync_copy(acc_sh.at[pl.ds(s * RPT, RPT)],
                    out_hbm.at[c, pl.ds(s * RPT, RPT)])


_SC_AGG_CACHE = []


def _sc_agg(*args):
    if not _SC_AGG_CACHE:
        _SC_AGG_CACHE.append(pl.kernel(
            _sc_agg_body,
            out_type=jax.ShapeDtypeStruct((NC, NPAD, H), _F32),
            mesh=plsc.VectorSubcoreMesh(core_axis_name="c",
                                        subcore_axis_name="s",
                                        num_cores=NC, num_subcores=NS),
            scratch_types=[
                pltpu.VMEM((NCHUNK, CH), jnp.int32),
                pltpu.VMEM((NCHUNK, CH), jnp.int32),
                pltpu.VMEM((NBUF, CH, H), _F32),
                pltpu.VMEM_SHARED((NPAD, H), _F32),
                pltpu.SemaphoreType.DMA,
                pltpu.SemaphoreType.DMA,
            ],
            compiler_params=pltpu.CompilerParams(use_tc_tiling_on_sc=False),
        ))
    return _SC_AGG_CACHE[0](*args)


def _mm_body(x_ref, wl_ref, wr_ref, o_ref):
    xe = x_ref[0:N:2, :]
    xo = x_ref[1:N:2, :]
    o_ref[...] = (
        lax.dot(xe, wl_ref[...], precision=_PREC, preferred_element_type=_F32)
        + lax.dot(xo, wr_ref[...], precision=_PREC,
                  preferred_element_type=_F32))


_mm = pl.pallas_call(
    _mm_body, out_shape=jax.ShapeDtypeStruct((NH, D), _F32))


def _fold_mean(s):
    m = s[:, :H] + s[:, H:]
    return jnp.concatenate([m, m], axis=1) * (1.0 / N)


def _bn_relu_packed(z, g2, be2):
    m = _fold_mean(jnp.sum(z, axis=0, keepdims=True))
    d = z - m
    v = _fold_mean(jnp.sum(d * d, axis=0, keepdims=True))
    return jnp.maximum(d * lax.rsqrt(v + 1e-5) * g2 + be2, 0.0)


def _zsum(q_ref, agg_ref, b2_ref):
    return (q_ref[...] + agg_ref[0:NH, :] + agg_ref[MROW:MROW + NH, :]
            + b2_ref[...])


def _mid_body(q_ref, agg_ref, b2_ref, g2_ref, be2_ref, wbd_ref, o_ref):
    z = _zsum(q_ref, agg_ref, b2_ref)
    h = _bn_relu_packed(z, g2_ref[...], be2_ref[...])
    o_ref[...] = lax.dot(h, wbd_ref[...], precision=_PREC,
                         preferred_element_type=_F32)


_mid = pl.pallas_call(
    _mid_body, out_shape=jax.ShapeDtypeStruct((NH, D), _F32))


def _fin_body(q_ref, agg_ref, b2_ref, g2_ref, be2_ref, be_ref, bo_ref,
              w1_ref, b1_ref, w2_ref, b2b_ref, o_ref):
    z = _zsum(q_ref, agg_ref, b2_ref)
    h = _bn_relu_packed(z, g2_ref[...], be2_ref[...])
    iota = lax.broadcasted_iota(jnp.int32, (NH, G), 1)
    ohe = (be_ref[...] == iota).astype(_F32)
    oho = (bo_ref[...] == iota).astype(_F32)
    dn = (((0,), (0,)), ((), ()))
    se = lax.dot_general(ohe, h, dn, precision=_PREC,
                         preferred_element_type=_F32)
    so = lax.dot_general(oho, h, dn, precision=_PREC,
                         preferred_element_type=_F32)
    sums = se[:, :H] + so[:, H:]
    ones = jnp.ones((NH, 1), _F32)
    cnt = (lax.dot_general(ohe, ones, dn, precision=_PREC,
                           preferred_element_type=_F32)
           + lax.dot_general(oho, ones, dn, precision=_PREC,
                             preferred_element_type=_F32))
    pooled = sums / jnp.maximum(cnt, 1.0)
    h2 = jnp.maximum(
        lax.dot(pooled, w1_ref[...], precision=_PREC,
                preferred_element_type=_F32) + b1_ref[...], 0.0)
    logit = jnp.sum(h2 * w2_ref[...], axis=1, keepdims=True) + b2b_ref[...]
    o_ref[...] = jax.nn.sigmoid(logit)


_fin = pl.pallas_call(
    _fin_body, out_shape=jax.ShapeDtypeStruct((G, 1), _F32))


def _pack2(v):
    return jnp.concatenate([v, v]).reshape(1, D)


def _blockdiag(w):
    zb = jnp.zeros((H, H), _F32)
    return jnp.concatenate(
        [jnp.concatenate([w, zb], axis=1),
         jnp.concatenate([zb, w], axis=1)], axis=0)


def kernel(x, params, edge_index, batch):
    src = edge_index[:, 0].reshape(NW, NCHUNK, CH)
    dst = edge_index[:, 1].reshape(NW, NCHUNK, CH)
    zeros = jnp.zeros((RPT, H), _F32)
    bh = batch.reshape(NH, 2)
    be2d = bh[:, 0].reshape(NH, 1)
    bo2d = bh[:, 1].reshape(NH, 1)
    zpad = jnp.zeros((D, H), _F32)
    w1l = jnp.concatenate([params['W1'], zpad], axis=1)
    w1r = jnp.concatenate([zpad, params['W1']], axis=1)

    q = _mm(x, w1l, w1r)
    for i in range(1, 6):
        aggs = _sc_agg(q.reshape(N, H), src, dst, zeros)
        aggp = aggs.reshape(NC * MROW, D)
        b2 = _pack2(params['b%d' % i])
        g2 = _pack2(params['g%d' % i])
        be2 = _pack2(params['be%d' % i])
        if i < 5:
            q = _mid(q, aggp, b2, g2, be2,
                     _blockdiag(params['W%d' % (i + 1)]))
        else:
            out = _fin(q, aggp, b2, g2, be2, be2d, bo2d,
                       params['fc1_W'], params['fc1_b'].reshape(1, H),
                       params['fc2_W'].reshape(1, H),
                       params['fc2_b'].reshape(1, 1))
    return out.reshape(G)


# speedup vs baseline: 17.5477x; 1.0170x over previous
import jax
import jax.numpy as jnp
from jax import lax
from jax.experimental import pallas as pl
from jax.experimental.pallas import tpu as pltpu
from jax.experimental.pallas import tpu_sc as plsc

N = 10000
E = 320000
D = 128
H = 64
G = 128

NC = 2
NS = 16
NW = NC * NS
CH = 125
NCHUNK = E // (NW * CH)
RPT = 640
NPAD = NS * RPT
NH = N // 2
MROW = NPAD // 2

_F32 = jnp.float32
_PREC = lax.Precision.HIGHEST


NBUF = 8
NOUTER = NCHUNK // NBUF


def _sc_agg_body(q_hbm, src_hbm, dst_hbm, zeros_hbm, out_hbm,
                 src_v, dst_v, rows_v, acc_sh, sem_g, sem_s):
    c = lax.axis_index("c")
    s = lax.axis_index("s")
    wid = c * NS + s

    z = pltpu.async_copy(zeros_hbm, acc_sh.at[pl.ds(s * RPT, RPT)], sem_g)
    a = pltpu.async_copy(src_hbm.at[wid], src_v, sem_g)
    b = pltpu.async_copy(dst_hbm.at[wid], dst_v, sem_g)
    z.wait()
    a.wait()
    b.wait()
    plsc.subcore_barrier()

    def _drain_scatter(b):
        pltpu.make_async_copy(q_hbm.at[pl.ds(0, CH)], rows_v.at[b],
                              sem_s).wait()

    def outer(i, carry):
        gathers = []
        for b in range(NBUF):
            @pl.when(i > 0)
            def _(b=b):
                _drain_scatter(b)
            gathers.append(pltpu.async_copy(
                q_hbm.at[src_v.at[i * NBUF + b]], rows_v.at[b], sem_g))
        for b in range(NBUF):
            gathers[b].wait()
            pltpu.async_copy(rows_v.at[b], acc_sh.at[dst_v.at[i * NBUF + b]],
                             sem_s, add=True)
        return carry

    lax.fori_loop(0, NOUTER, outer, 0)
    for b in range(NBUF):
        _drain_scatter(b)

    plsc.subcore_barrier()
    pltpu.sync_copy(acc_sh.at[pl.ds(s * RPT, RPT)],
                    out_hbm.at[c, pl.ds(s * RPT, RPT)])


_SC_AGG_CACHE = []


def _sc_agg(*args):
    if not _SC_AGG_CACHE:
        _SC_AGG_CACHE.append(pl.kernel(
            _sc_agg_body,
            out_type=jax.ShapeDtypeStruct((NC, NPAD, H), _F32),
            mesh=plsc.VectorSubcoreMesh(core_axis_name="c",
                                        subcore_axis_name="s",
                                        num_cores=NC, num_subcores=NS),
            scratch_types=[
                pltpu.VMEM((NCHUNK, CH), jnp.int32),
                pltpu.VMEM((NCHUNK, CH), jnp.int32),
                pltpu.VMEM((NBUF, CH, H), _F32),
                pltpu.VMEM_SHARED((NPAD, H), _F32),
                pltpu.SemaphoreType.DMA,
                pltpu.SemaphoreType.DMA,
            ],
            compiler_params=pltpu.CompilerParams(use_tc_tiling_on_sc=False),
        ))
    return _SC_AGG_CACHE[0](*args)


def _mm_body(x_ref, wl_ref, wr_ref, o_ref):
    xe = x_ref[0:N:2, :]
    xo = x_ref[1:N:2, :]
    o_ref[...] = (
        lax.dot(xe, wl_ref[...], precision=_PREC, preferred_element_type=_F32)
        + lax.dot(xo, wr_ref[...], precision=_PREC,
                  preferred_element_type=_F32))


_mm = pl.pallas_call(
    _mm_body, out_shape=jax.ShapeDtypeStruct((NH, D), _F32))


def _fold_mean(s):
    m = s[:, :H] + s[:, H:]
    return jnp.concatenate([m, m], axis=1) * (1.0 / N)


def _bn_relu_packed(z, g2, be2):
    m = _fold_mean(jnp.sum(z, axis=0, keepdims=True))
    d = z - m
    v = _fold_mean(jnp.sum(d * d, axis=0, keepdims=True))
    return jnp.maximum(d * lax.rsqrt(v + 1e-5) * g2 + be2, 0.0)


def _zsum(q_ref, agg_ref, b2_ref):
    return (q_ref[...] + agg_ref[0:NH, :] + agg_ref[MROW:MROW + NH, :]
            + b2_ref[...])


def _mid_body(q_ref, agg_ref, b2_ref, g2_ref, be2_ref, wbd_ref, o_ref):
    z = _zsum(q_ref, agg_ref, b2_ref)
    h = _bn_relu_packed(z, g2_ref[...], be2_ref[...])
    o_ref[...] = lax.dot(h, wbd_ref[...], precision=_PREC,
                         preferred_element_type=_F32)


_mid = pl.pallas_call(
    _mid_body, out_shape=jax.ShapeDtypeStruct((NH, D), _F32))


def _fin_body(q_ref, agg_ref, b2_ref, g2_ref, be2_ref, be_ref, bo_ref,
              w1_ref, b1_ref, w2_ref, b2b_ref, o_ref):
    z = _zsum(q_ref, agg_ref, b2_ref)
    h = _bn_relu_packed(z, g2_ref[...], be2_ref[...])
    iota = lax.broadcasted_iota(jnp.int32, (NH, G), 1)
    ohe = (be_ref[...] == iota).astype(_F32)
    oho = (bo_ref[...] == iota).astype(_F32)
    dn = (((0,), (0,)), ((), ()))
    se = lax.dot_general(ohe, h, dn, precision=_PREC,
                         preferred_element_type=_F32)
    so = lax.dot_general(oho, h, dn, precision=_PREC,
                         preferred_element_type=_F32)
    sums = se[:, :H] + so[:, H:]
    ones = jnp.ones((NH, 1), _F32)
    cnt = (lax.dot_general(ohe, ones, dn, precision=_PREC,
                           preferred_element_type=_F32)
           + lax.dot_general(oho, ones, dn, precision=_PREC,
                             preferred_element_type=_F32))
    pooled = sums / jnp.maximum(cnt, 1.0)
    h2 = jnp.maximum(
        lax.dot(pooled, w1_ref[...], precision=_PREC,
                preferred_element_type=_F32) + b1_ref[...], 0.0)
    logit = jnp.sum(h2 * w2_ref[...], axis=1, keepdims=True) + b2b_ref[...]
    o_ref[...] = jax.nn.sigmoid(logit)


_fin = pl.pallas_call(
    _fin_body, out_shape=jax.ShapeDtypeStruct((G, 1), _F32))


def _pack2(v):
    return jnp.concatenate([v, v]).reshape(1, D)


def _blockdiag(w):
    zb = jnp.zeros((H, H), _F32)
    return jnp.concatenate(
        [jnp.concatenate([w, zb], axis=1),
         jnp.concatenate([zb, w], axis=1)], axis=0)


def kernel(x, params, edge_index, batch):
    src = edge_index[:, 0].reshape(NW, NCHUNK, CH)
    dst = edge_index[:, 1].reshape(NW, NCHUNK, CH)
    zeros = jnp.zeros((RPT, H), _F32)
    bh = batch.reshape(NH, 2)
    be2d = bh[:, 0].reshape(NH, 1)
    bo2d = bh[:, 1].reshape(NH, 1)
    zpad = jnp.zeros((D, H), _F32)
    w1l = jnp.concatenate([params['W1'], zpad], axis=1)
    w1r = jnp.concatenate([zpad, params['W1']], axis=1)

    q = _mm(x, w1l, w1r)
    for i in range(1, 6):
        aggs = _sc_agg(q.reshape(N, H), src, dst, zeros)
        aggp = aggs.reshape(NC * MROW, D)
        b2 = _pack2(params['b%d' % i])
        g2 = _pack2(params['g%d' % i])
        be2 = _pack2(params['be%d' % i])
        if i < 5:
            q = _mid(q, aggp, b2, g2, be2,
                     _blockdiag(params['W%d' % (i + 1)]))
        else:
            out = _fin(q, aggp, b2, g2, be2, be2d, bo2d,
                       params['fc1_W'], params['fc1_b'].reshape(1, H),
                       params['fc2_W'].reshape(1, H),
                       params['fc2_b'].reshape(1, 1))
    return out.reshape(G)


# speedup vs baseline: 17.8464x; 1.0170x over previous
import jax
import jax.numpy as jnp
from jax import lax
from jax.experimental import pallas as pl
from jax.experimental.pallas import tpu as pltpu
from jax.experimental.pallas import tpu_sc as plsc

N = 10000
E = 320000
D = 128
H = 64
G = 128

NC = 2
NS = 16
NW = NC * NS
CH = 125
NCHUNK = E // (NW * CH)
RPT = 625
NPAD = NS * RPT
NH = N // 2
MROW = NPAD // 2

_F32 = jnp.float32
_PREC = lax.Precision.HIGHEST


NBUF = 8
NOUTER = NCHUNK // NBUF


def _sc_agg_body(q_hbm, src_hbm, dst_hbm, zeros_hbm, out_hbm,
                 src_v, dst_v, rows_v, acc_sh, sem_g, sem_s):
    c = lax.axis_index("c")
    s = lax.axis_index("s")
    wid = c * NS + s

    z = pltpu.async_copy(zeros_hbm, acc_sh.at[pl.ds(s * RPT, RPT)], sem_g)
    a = pltpu.async_copy(src_hbm.at[wid], src_v, sem_g)
    b = pltpu.async_copy(dst_hbm.at[wid], dst_v, sem_g)
    z.wait()
    a.wait()
    b.wait()
    plsc.subcore_barrier()

    def _drain_scatter(b):
        pltpu.make_async_copy(q_hbm.at[pl.ds(0, CH)], rows_v.at[b],
                              sem_s).wait()

    def outer(i, carry):
        gathers = []
        for b in range(NBUF):
            @pl.when(i > 0)
            def _(b=b):
                _drain_scatter(b)
            gathers.append(pltpu.async_copy(
                q_hbm.at[src_v.at[i * NBUF + b]], rows_v.at[b], sem_g))
        for b in range(NBUF):
            gathers[b].wait()
            pltpu.async_copy(rows_v.at[b], acc_sh.at[dst_v.at[i * NBUF + b]],
                             sem_s, add=True)
        return carry

    lax.fori_loop(0, NOUTER, outer, 0)
    for b in range(NBUF):
        _drain_scatter(b)

    plsc.subcore_barrier()
    pltpu.sync_copy(acc_sh.at[pl.ds(s * RPT, RPT)],
                    out_hbm.at[c, pl.ds(s * RPT, RPT)])


_SC_AGG_CACHE = []


def _sc_agg(*args):
    if not _SC_AGG_CACHE:
        _SC_AGG_CACHE.append(pl.kernel(
            _sc_agg_body,
            out_type=jax.ShapeDtypeStruct((NC, NPAD, H), _F32),
            mesh=plsc.VectorSubcoreMesh(core_axis_name="c",
                                        subcore_axis_name="s",
                                        num_cores=NC, num_subcores=NS),
            scratch_types=[
                pltpu.VMEM((NCHUNK, CH), jnp.int32),
                pltpu.VMEM((NCHUNK, CH), jnp.int32),
                pltpu.VMEM((NBUF, CH, H), _F32),
                pltpu.VMEM_SHARED((NPAD, H), _F32),
                pltpu.SemaphoreType.DMA,
                pltpu.SemaphoreType.DMA,
            ],
            compiler_params=pltpu.CompilerParams(use_tc_tiling_on_sc=False),
        ))
    return _SC_AGG_CACHE[0](*args)


def _mm_body(x_ref, wl_ref, wr_ref, o_ref):
    xe = x_ref[0:N:2, :]
    xo = x_ref[1:N:2, :]
    o_ref[...] = (
        lax.dot(xe, wl_ref[...], precision=_PREC, preferred_element_type=_F32)
        + lax.dot(xo, wr_ref[...], precision=_PREC,
                  preferred_element_type=_F32))


_mm = pl.pallas_call(
    _mm_body, out_shape=jax.ShapeDtypeStruct((NH, D), _F32))


def _fold_mean(s):
    m = s[:, :H] + s[:, H:]
    return jnp.concatenate([m, m], axis=1) * (1.0 / N)


def _bn_relu_packed(z, g2, be2):
    m = _fold_mean(jnp.sum(z, axis=0, keepdims=True))
    v = _fold_mean(jnp.sum(z * z, axis=0, keepdims=True)) - m * m
    return jnp.maximum((z - m) * lax.rsqrt(v + 1e-5) * g2 + be2, 0.0)


def _zsum(q_ref, agg_ref, b2_ref):
    return (q_ref[...] + agg_ref[0:NH, :] + agg_ref[MROW:MROW + NH, :]
            + b2_ref[...])


def _mid_body(q_ref, agg_ref, b2_ref, g2_ref, be2_ref, wbd_ref, o_ref):
    z = _zsum(q_ref, agg_ref, b2_ref)
    h = _bn_relu_packed(z, g2_ref[...], be2_ref[...])
    o_ref[...] = lax.dot(h, wbd_ref[...], precision=_PREC,
                         preferred_element_type=_F32)


_mid = pl.pallas_call(
    _mid_body, out_shape=jax.ShapeDtypeStruct((NH, D), _F32))


def _fin_body(q_ref, agg_ref, b2_ref, g2_ref, be2_ref, be_ref, bo_ref,
              w1_ref, b1_ref, w2_ref, b2b_ref, o_ref):
    z = _zsum(q_ref, agg_ref, b2_ref)
    h = _bn_relu_packed(z, g2_ref[...], be2_ref[...])
    iota = lax.broadcasted_iota(jnp.int32, (NH, G), 1)
    ohe = (be_ref[...] == iota).astype(_F32)
    oho = (bo_ref[...] == iota).astype(_F32)
    dn = (((0,), (0,)), ((), ()))
    se = lax.dot_general(ohe, h, dn, precision=_PREC,
                         preferred_element_type=_F32)
    so = lax.dot_general(oho, h, dn, precision=_PREC,
                         preferred_element_type=_F32)
    sums = se[:, :H] + so[:, H:]
    ones = jnp.ones((NH, 1), _F32)
    cnt = (lax.dot_general(ohe, ones, dn, precision=_PREC,
                           preferred_element_type=_F32)
           + lax.dot_general(oho, ones, dn, precision=_PREC,
                             preferred_element_type=_F32))
    pooled = sums / jnp.maximum(cnt, 1.0)
    h2 = jnp.maximum(
        lax.dot(pooled, w1_ref[...], precision=_PREC,
                preferred_element_type=_F32) + b1_ref[...], 0.0)
    logit = jnp.sum(h2 * w2_ref[...], axis=1, keepdims=True) + b2b_ref[...]
    o_ref[...] = jax.nn.sigmoid(logit)


_fin = pl.pallas_call(
    _fin_body, out_shape=jax.ShapeDtypeStruct((G, 1), _F32))


def _pack2(v):
    return jnp.concatenate([v, v]).reshape(1, D)


def _blockdiag(w):
    zb = jnp.zeros((H, H), _F32)
    return jnp.concatenate(
        [jnp.concatenate([w, zb], axis=1),
         jnp.concatenate([zb, w], axis=1)], axis=0)


def kernel(x, params, edge_index, batch):
    src = edge_index[:, 0].reshape(NW, NCHUNK, CH)
    dst = edge_index[:, 1].reshape(NW, NCHUNK, CH)
    zeros = jnp.zeros((RPT, H), _F32)
    bh = batch.reshape(NH, 2)
    be2d = bh[:, 0].reshape(NH, 1)
    bo2d = bh[:, 1].reshape(NH, 1)
    zpad = jnp.zeros((D, H), _F32)
    w1l = jnp.concatenate([params['W1'], zpad], axis=1)
    w1r = jnp.concatenate([zpad, params['W1']], axis=1)

    q = _mm(x, w1l, w1r)
    for i in range(1, 6):
        aggs = _sc_agg(q.reshape(N, H), src, dst, zeros)
        aggp = aggs.reshape(NC * MROW, D)
        b2 = _pack2(params['b%d' % i])
        g2 = _pack2(params['g%d' % i])
        be2 = _pack2(params['be%d' % i])
        if i < 5:
            q = _mid(q, aggp, b2, g2, be2,
                     _blockdiag(params['W%d' % (i + 1)]))
        else:
            out = _fin(q, aggp, b2, g2, be2, be2d, bo2d,
                       params['fc1_W'], params['fc1_b'].reshape(1, H),
                       params['fc2_W'].reshape(1, H),
                       params['fc2_b'].reshape(1, 1))
    return out.reshape(G)
